# baseline (speedup 1.0000x reference)
import jax
import jax.numpy as jnp
from jax.experimental import pallas as pl
from jax.experimental.pallas import tpu as pltpu


def _gcn_body(x_ref, w_ref, b_ref, adj_ref, out_ref, supp_ref):
    b = pl.program_id(0)
    i = pl.program_id(1)
    nb = pl.num_programs(0) - 1
    blk = x_ref.shape[1]

    @pl.when(b < nb)
    def _():
        p = jax.lax.rem(b, 2)
        supp_ref[p, pl.ds(i * blk, blk), :] = jnp.dot(
            x_ref[0].astype(jnp.bfloat16),
            w_ref[...].astype(jnp.bfloat16),
            preferred_element_type=jnp.float32,
        ).astype(jnp.bfloat16)

    @pl.when(b > 0)
    def _():
        q = jax.lax.rem(b + 1, 2)
        out_ref[0] = (
            jnp.dot(
                adj_ref[0].astype(jnp.bfloat16),
                supp_ref[q],
                preferred_element_type=jnp.float32,
            )
            + b_ref[...]
        )


def kernel(input, adj, weight, bias):
    B, N, IN = input.shape
    OUT = weight.shape[1]
    BLK_I = min(1024, N)

    out = pl.pallas_call(
        _gcn_body,
        grid=(B + 1, N // BLK_I),
        in_specs=[
            pl.BlockSpec(
                (1, BLK_I, IN),
                lambda b, i: (
                    jnp.minimum(b, pl.num_programs(0) - 2),
                    jnp.where(b == pl.num_programs(0) - 1, 0, i),
                    0,
                ),
            ),
            pl.BlockSpec((IN, OUT), lambda b, i: (0, 0)),
            pl.BlockSpec((1, OUT), lambda b, i: (0, 0)),
            pl.BlockSpec(
                (1, BLK_I, N),
                lambda b, i: (
                    jnp.maximum(b - 1, 0),
                    jnp.where(b == 0, 0, i),
                    0,
                ),
            ),
        ],
        out_specs=pl.BlockSpec(
            (1, BLK_I, OUT),
            lambda b, i: (
                jnp.maximum(b - 1, 0),
                jnp.where(b == 0, 0, i),
                0,
            ),
        ),
        out_shape=jax.ShapeDtypeStruct((B, N, OUT), jnp.float32),
        scratch_shapes=[pltpu.VMEM((2, N, OUT), jnp.bfloat16)],
        compiler_params=pltpu.CompilerParams(
            vmem_limit_bytes=64 * 1024 * 1024,
        ),
    )(input, weight, bias.reshape(1, OUT), adj)
    return out
